# baseline (speedup 1.0000x reference)
import functools

import jax
import jax.numpy as jnp
from jax import lax
from jax.experimental import pallas as pl
from jax.experimental.pallas import tpu as pltpu
from jax.experimental.pallas import tpu_sc as plsc

N = 10000
NP = 10240
E = 320000
D = 128
H = 16
DH = 8
B = 128
L = 50

NC = 2
NS = 16
NW = NC * NS

CH = 128
NCH_W = 160
CH0_W = 144
CH1_W = 2 * NCH_W - CH0_W
ETOT = NW * NCH_W * CH
ECHUNKS = ETOT // CH

HTOT = 655360
HCHUNKS_PER_W = HTOT // 128 // NW

GTOT = 8192
GCHUNKS_PER_W = GTOT // 128 // NW

ROWS_PER_TILE = NP // NS

_mesh = plsc.VectorSubcoreMesh(
    core_axis_name="c", subcore_axis_name="s", num_cores=NC, num_subcores=NS)


def _zero_vmem_2d(ref, nrows):
  def row(r, _):
    for j in range(8):
      ref[r, pl.ds(j * 16, 16)] = jnp.zeros((16,), jnp.float32)
    return _
  lax.fori_loop(0, nrows, row, None)


def _hist_body(nodes_hbm, out_hbm, deg_sh, idx_all, ones_v, zbuf_v, sem):
  c = lax.axis_index("c")
  s = lax.axis_index("s")
  wid = c * NS + s
  pltpu.sync_copy(nodes_hbm.at[pl.ds(wid * HCHUNKS_PER_W, HCHUNKS_PER_W)],
                  idx_all)
  for j in range(8):
    ones_v[pl.ds(j * 16, 16)] = jnp.ones((16,), jnp.float32)
    zbuf_v[pl.ds(j * 16, 16)] = jnp.zeros((16,), jnp.float32)
  for k in range(ROWS_PER_TILE // 128):
    pltpu.sync_copy(zbuf_v, deg_sh.at[pl.ds(s * ROWS_PER_TILE + k * 128, 128)])
  plsc.subcore_barrier()

  def group(g, carry):
    for k in range(8):
      pltpu.async_copy(ones_v, deg_sh.at[idx_all.at[g * 8 + k]], sem, add=True)
    for k in range(8):
      pltpu.make_async_copy(ones_v, deg_sh.at[idx_all.at[g * 8 + k]],
                            sem).wait()
    return carry
  lax.fori_loop(0, HCHUNKS_PER_W // 8, group, None)
  plsc.subcore_barrier()
  pltpu.sync_copy(deg_sh.at[pl.ds(s * ROWS_PER_TILE, ROWS_PER_TILE)],
                  out_hbm.at[pl.ds(c * NP + s * ROWS_PER_TILE, ROWS_PER_TILE)])


_hist = pl.kernel(
    _hist_body,
    out_type=jax.ShapeDtypeStruct((NC * NP,), jnp.float32),
    mesh=_mesh,
    scratch_types=[
        pltpu.VMEM_SHARED((NP,), jnp.float32),
        pltpu.VMEM((HCHUNKS_PER_W, 128), jnp.int32),
        pltpu.VMEM((128,), jnp.float32),
        pltpu.VMEM((128,), jnp.float32),
        pltpu.SemaphoreType.DMA,
    ],
)


def _spmm_body(n1_hbm, n2_hbm, dv_hbm, h_hbm, out_hbm, acc_sh,
               n1b0, n1b1, n2b0, n2b1, dvb0, dvb1,
               rows_v0, rows_v1,
               gsem0, gsem1, ssem0, ssem1,
               asem0, asem1, bsem0, bsem1, dsem0, dsem1):
  c = lax.axis_index("c")
  s = lax.axis_index("s")
  n_my = jnp.where(c == 0, CH0_W, CH1_W)
  t0 = jnp.where(c == 0, s * CH0_W, NS * CH0_W + s * CH1_W)
  _zero_vmem_2d(rows_v0, CH)
  for k in range(ROWS_PER_TILE // CH):
    pltpu.sync_copy(rows_v0, acc_sh.at[pl.ds(s * ROWS_PER_TILE + k * CH, CH)])

  rows = (rows_v0, rows_v1)
  n1b = (n1b0, n1b1)
  n2b = (n2b0, n2b1)
  dvb = (dvb0, dvb1)
  gsem = (gsem0, gsem1)
  ssem = (ssem0, ssem1)
  asem = (asem0, asem1)
  bsem = (bsem0, bsem1)
  dsem = (dsem0, dsem1)

  def start_n1(ch, b):
    pltpu.async_copy(n1_hbm.at[t0 + ch], n1b[b], asem[b])

  def wait_n1(b):
    pltpu.make_async_copy(n1_hbm.at[0], n1b[b], asem[b]).wait()

  def start_n2(ch, b):
    pltpu.async_copy(n2_hbm.at[t0 + ch], n2b[b], bsem[b])

  def wait_n2(b):
    pltpu.make_async_copy(n2_hbm.at[0], n2b[b], bsem[b]).wait()

  def start_dv(ch, b):
    pltpu.async_copy(dv_hbm.at[t0 + ch], dvb[b], dsem[b])

  def wait_dv(b):
    pltpu.make_async_copy(dv_hbm.at[0], dvb[b], dsem[b]).wait()

  def start_gather(b):
    pltpu.async_copy(h_hbm.at[n2b[b]], rows[b], gsem[b])

  def wait_gather(b):
    pltpu.make_async_copy(h_hbm.at[n2b[b]], rows[b], gsem[b]).wait()

  def start_scatter(b):
    pltpu.async_copy(rows[b], acc_sh.at[n1b[b]], ssem[b], add=True)

  def wait_scatter(b):
    pltpu.make_async_copy(rows[b], acc_sh.at[n1b[b]], ssem[b]).wait()

  def scale(b):
    r = rows[b]
    for gq in range(CH // 16):
      dvv = dvb[b][pl.ds(gq * 16, 16)]
      wg = jnp.exp(-(dvv * dvv))
      for t in range(16):
        w = wg[t]
        e = gq * 16 + t
        for j in range(8):
          sl = pl.ds(j * 16, 16)
          r[e, sl] = r[e, sl] * w

  nit = n_my // 2
  plsc.subcore_barrier()
  for b in range(2):
    start_n2(b, b)
    start_dv(b, b)
    start_n1(b, b)
  for b in range(2):
    wait_n2(b)
    start_gather(b)

  def body(i, carry):
    c0 = 2 * i
    more = i < nit - 1

    wait_gather(0)

    @pl.when(more)
    def _():
      start_n2(c0 + 2, 0)
    wait_dv(0)
    scale(0)

    @pl.when(more)
    def _():
      start_dv(c0 + 2, 0)
    wait_n1(0)
    start_scatter(0)

    wait_gather(1)

    @pl.when(more)
    def _():
      start_n2(c0 + 3, 1)
    wait_dv(1)
    scale(1)

    @pl.when(more)
    def _():
      start_dv(c0 + 3, 1)
    wait_n1(1)
    start_scatter(1)

    wait_scatter(0)

    @pl.when(more)
    def _():
      start_n1(c0 + 2, 0)
      wait_n2(0)
      start_gather(0)
    wait_scatter(1)

    @pl.when(more)
    def _():
      start_n1(c0 + 3, 1)
      wait_n2(1)
      start_gather(1)
    return carry
  lax.fori_loop(0, nit, body, None)

  plsc.subcore_barrier()
  for k in range(ROWS_PER_TILE // CH):
    r = s * ROWS_PER_TILE + k * CH
    pltpu.sync_copy(acc_sh.at[pl.ds(r, CH)], out_hbm.at[pl.ds(c * NP + r, CH)])


_spmm = pl.kernel(
    _spmm_body,
    out_type=jax.ShapeDtypeStruct((NC * NP, D), jnp.float32),
    mesh=_mesh,
    scratch_types=(
        [pltpu.VMEM_SHARED((NP, D), jnp.float32)]
        + [pltpu.VMEM((CH,), jnp.int32) for _ in range(4)]
        + [pltpu.VMEM((CH,), jnp.float32) for _ in range(2)]
        + [pltpu.VMEM((CH, D), jnp.float32) for _ in range(2)]
        + [pltpu.SemaphoreType.DMA for _ in range(10)]
    ),
)


def _gather_body(idx_hbm, enc_hbm, out_hbm, idx_v0, idx_v1, rows_v0, rows_v1,
                 sem0, sem1, wsem0, wsem1):
  c = lax.axis_index("c")
  s = lax.axis_index("s")
  wid = c * NS + s
  base0 = wid * (GCHUNKS_PER_W * 128)
  base1 = base0 + 128
  pltpu.sync_copy(idx_hbm.at[pl.ds(base0, 128)], idx_v0)
  pltpu.async_copy(enc_hbm.at[idx_v0], rows_v0, sem0)
  pltpu.sync_copy(idx_hbm.at[pl.ds(base1, 128)], idx_v1)
  pltpu.async_copy(enc_hbm.at[idx_v1], rows_v1, sem1)
  pltpu.make_async_copy(enc_hbm.at[idx_v0], rows_v0, sem0).wait()
  pltpu.async_copy(rows_v0, out_hbm.at[pl.ds(base0, 128)], wsem0)
  pltpu.make_async_copy(enc_hbm.at[idx_v1], rows_v1, sem1).wait()
  pltpu.async_copy(rows_v1, out_hbm.at[pl.ds(base1, 128)], wsem1)
  pltpu.make_async_copy(rows_v0, out_hbm.at[pl.ds(base0, 128)], wsem0).wait()
  pltpu.make_async_copy(rows_v1, out_hbm.at[pl.ds(base1, 128)], wsem1).wait()


_gather = pl.kernel(
    _gather_body,
    out_type=jax.ShapeDtypeStruct((GTOT, D), jnp.float32),
    mesh=_mesh,
    scratch_types=[
        pltpu.VMEM((128,), jnp.int32),
        pltpu.VMEM((128,), jnp.int32),
        pltpu.VMEM((128, D), jnp.float32),
        pltpu.VMEM((128, D), jnp.float32),
        pltpu.SemaphoreType.DMA,
        pltpu.SemaphoreType.DMA,
        pltpu.SemaphoreType.DMA,
        pltpu.SemaphoreType.DMA,
    ],
)


_RB = 512
_GRID = NP // _RB


def _mm_scale_body(x_ref, dp_ref, w_ref, b_ref, o_ref):
  dp = dp_ref[...]
  sc = lax.rsqrt(dp[0] + dp[1] + 1.0)
  h = lax.dot_general(x_ref[...], w_ref[...], (((1,), (1,)), ((), ())),
                      preferred_element_type=jnp.float32)
  h = (h + b_ref[...]) * sc[:, None]
  rows = pl.program_id(0) * _RB + lax.broadcasted_iota(jnp.int32, h.shape, 0)
  o_ref[...] = jnp.where(rows < N, h, 0.0)


def _lrelu_norm(u):
  e = jnp.where(u >= 0, u, 0.01 * u)
  nrm = jnp.sqrt(jnp.sum(e * e, axis=1, keepdims=True))
  return e / jnp.maximum(nrm, 1e-12)


def _fuse_body(p0_ref, p1_ref, dp_ref, w_ref, b_ref, o_ref):
  dp = dp_ref[...]
  sc = lax.rsqrt(dp[0] + dp[1] + 1.0)
  e = _lrelu_norm(p0_ref[...] + p1_ref[...])
  h = lax.dot_general(e, w_ref[...], (((1,), (1,)), ((), ())),
                      preferred_element_type=jnp.float32)
  h = (h + b_ref[...]) * sc[:, None]
  rows = pl.program_id(0) * _RB + lax.broadcasted_iota(jnp.int32, h.shape, 0)
  o_ref[...] = jnp.where(rows < N, h, 0.0)


def _norm_body(p0_ref, p1_ref, o_ref):
  o_ref[...] = _lrelu_norm(p0_ref[...] + p1_ref[...])


def _attn_body(x_ref, wqkv_ref, bqkv_ref, o_ref):
  x = x_ref[0]
  qkv = lax.dot_general(x, wqkv_ref[...], (((1,), (1,)), ((), ())),
                        preferred_element_type=jnp.float32) + bqkv_ref[...]
  q = qkv[:, :D]
  k = qkv[:, D:2 * D]
  v = qkv[:, 2 * D:]
  lane = lax.broadcasted_iota(jnp.int32, (1, D), 1) // DH
  scale = 1.0 / jnp.sqrt(jnp.float32(DH))
  acc = jnp.zeros((1, D), jnp.float32)
  for h in range(H):
    mh = (lane == h).astype(jnp.float32)
    sco = lax.dot_general(q * mh, k, (((1,), (1,)), ((), ())),
                          preferred_element_type=jnp.float32) * scale
    sco = sco - jnp.max(sco, axis=1, keepdims=True)
    ex = jnp.exp(sco)
    p = ex / jnp.sum(ex, axis=1, keepdims=True)
    m = jnp.mean(p, axis=0, keepdims=True)
    acc = acc + lax.dot_general(m, v * mh, (((1,), (0,)), ((), ())),
                                preferred_element_type=jnp.float32)
  o_ref[0] = acc


def _proj_body(x_ref, w_ref, b_ref, o_ref):
  o_ref[...] = lax.dot_general(x_ref[...], w_ref[...], (((1,), (1,)), ((), ())),
                               preferred_element_type=jnp.float32) + b_ref[...]


def _full(shape):
  return pl.BlockSpec(shape, lambda b: tuple(0 for _ in shape))


_mm_scale = pl.pallas_call(
    _mm_scale_body,
    grid=(_GRID,),
    in_specs=[
        pl.BlockSpec((_RB, D), lambda b: (b, 0)),
        pl.BlockSpec((2, _RB), lambda b: (0, b)),
        _full((D, D)),
        _full((1, D)),
    ],
    out_specs=pl.BlockSpec((_RB, D), lambda b: (b, 0)),
    out_shape=jax.ShapeDtypeStruct((NP, D), jnp.float32),
)

_fuse = pl.pallas_call(
    _fuse_body,
    grid=(_GRID,),
    in_specs=[
        pl.BlockSpec((_RB, D), lambda b: (b, 0)),
        pl.BlockSpec((_RB, D), lambda b: (b + _GRID, 0)),
        pl.BlockSpec((2, _RB), lambda b: (0, b)),
        _full((D, D)),
        _full((1, D)),
    ],
    out_specs=pl.BlockSpec((_RB, D), lambda b: (b, 0)),
    out_shape=jax.ShapeDtypeStruct((NP, D), jnp.float32),
)

_norm = pl.pallas_call(
    _norm_body,
    grid=(_GRID,),
    in_specs=[
        pl.BlockSpec((_RB, D), lambda b: (b, 0)),
        pl.BlockSpec((_RB, D), lambda b: (b + _GRID, 0)),
    ],
    out_specs=pl.BlockSpec((_RB, D), lambda b: (b, 0)),
    out_shape=jax.ShapeDtypeStruct((NP, D), jnp.float32),
)

_attn = pl.pallas_call(
    _attn_body,
    grid=(B,),
    in_specs=[
        pl.BlockSpec((1, L, D), lambda b: (b, 0, 0)),
        _full((3 * D, D)),
        _full((1, 3 * D)),
    ],
    out_specs=pl.BlockSpec((1, 1, D), lambda b: (b, 0, 0)),
    out_shape=jax.ShapeDtypeStruct((B, 1, D), jnp.float32),
)

_proj = pl.pallas_call(
    _proj_body,
    grid=(1,),
    in_specs=[_full((B, D)), _full((D, D)), _full((1, D))],
    out_specs=_full((B, D)),
    out_shape=jax.ShapeDtypeStruct((B, D), jnp.float32),
)


def kernel(poi_table, dist_edges, dist_vec, data_x, data_poi,
           W0, b0, W1, b1, Wqkv, bqkv, Wo, bo):
  i32 = jnp.int32
  f32 = jnp.float32
  d0 = dist_edges[0].astype(i32)
  d1 = dist_edges[1].astype(i32)
  loop = jnp.arange(N, dtype=i32)
  epad = ETOT - (2 * E + N)
  pad_idx = jnp.full((epad,), NP - 1, i32)
  n1 = jnp.concatenate([d0, d1, loop, pad_idx])
  n2 = jnp.concatenate([d1, d0, loop, pad_idx])
  dv = jnp.concatenate([dist_vec.astype(f32), dist_vec.astype(f32),
                        jnp.zeros((N,), f32), jnp.full((epad,), 1e9, f32)])
  n1c = n1.reshape(ECHUNKS, CH)
  n2c = n2.reshape(ECHUNKS, CH)
  dv2 = dv.reshape(ECHUNKS, CH)
  nodes = jnp.concatenate(
      [d0, d1, jnp.full((HTOT - 2 * E,), NP - 1, i32)]).reshape(HTOT // 128, 128)

  degp = _hist(nodes).reshape(NC, NP)
  ptab = jnp.pad(poi_table.astype(f32), ((0, NP - N), (0, 0)))
  b0r = b0.reshape(1, D).astype(f32)
  b1r = b1.reshape(1, D).astype(f32)

  h1 = _mm_scale(ptab, degp, W0.astype(f32), b0r)
  p1 = _spmm(n1c, n2c, dv2, h1)
  h2 = _fuse(p1, p1, degp, W1.astype(f32), b1r)
  p2 = _spmm(n1c, n2c, dv2, h2)
  enc = _norm(p2, p2)

  gidx = jnp.concatenate([data_x.astype(i32), data_poi.astype(i32),
                          jnp.zeros((GTOT - B * L - B,), i32)])
  g = _gather(gidx, enc)
  poi_embed = g[B * L:B * L + B]
  seq3 = g[:B * L].reshape(B, L, D)
  ctxm = _attn(seq3, Wqkv.astype(f32),
               bqkv.reshape(1, 3 * D).astype(f32)).reshape(B, D)
  aggr = _proj(ctxm, Wo.astype(f32), bo.reshape(1, D).astype(f32))
  return (aggr, poi_embed)


# speedup vs baseline: 1.0788x; 1.0788x over previous
import functools

import jax
import jax.numpy as jnp
from jax import lax
from jax.experimental import pallas as pl
from jax.experimental.pallas import tpu as pltpu
from jax.experimental.pallas import tpu_sc as plsc

N = 10000
NP = 10240
E = 320000
D = 128
H = 16
DH = 8
B = 128
L = 50

NC = 2
NS = 16
NW = NC * NS

CH = 128
NCH_W = 160
CH0_W = 176
CH1_W = 2 * NCH_W - CH0_W
ETOT = NW * NCH_W * CH
ECHUNKS = ETOT // CH

HTOT = 655360
HCHUNKS_PER_W = HTOT // 128 // NW

GTOT = 8192
GCHUNKS_PER_W = GTOT // 128 // NW

ROWS_PER_TILE = NP // NS

_mesh = plsc.VectorSubcoreMesh(
    core_axis_name="c", subcore_axis_name="s", num_cores=NC, num_subcores=NS)


def _zero_vmem_2d(ref, nrows):
  def row(r, _):
    for j in range(8):
      ref[r, pl.ds(j * 16, 16)] = jnp.zeros((16,), jnp.float32)
    return _
  lax.fori_loop(0, nrows, row, None)


def _hist_body(nodes_hbm, out_hbm, deg_sh, idx_all, ones_v, zbuf_v, sem):
  c = lax.axis_index("c")
  s = lax.axis_index("s")
  wid = c * NS + s
  pltpu.sync_copy(nodes_hbm.at[pl.ds(wid * HCHUNKS_PER_W, HCHUNKS_PER_W)],
                  idx_all)
  for j in range(8):
    ones_v[pl.ds(j * 16, 16)] = jnp.ones((16,), jnp.float32)
    zbuf_v[pl.ds(j * 16, 16)] = jnp.zeros((16,), jnp.float32)
  for k in range(ROWS_PER_TILE // 128):
    pltpu.sync_copy(zbuf_v, deg_sh.at[pl.ds(s * ROWS_PER_TILE + k * 128, 128)])
  plsc.subcore_barrier()

  def group(g, carry):
    for k in range(8):
      pltpu.async_copy(ones_v, deg_sh.at[idx_all.at[g * 8 + k]], sem, add=True)
    for k in range(8):
      pltpu.make_async_copy(ones_v, deg_sh.at[idx_all.at[g * 8 + k]],
                            sem).wait()
    return carry
  lax.fori_loop(0, HCHUNKS_PER_W // 8, group, None)
  plsc.subcore_barrier()
  pltpu.sync_copy(deg_sh.at[pl.ds(s * ROWS_PER_TILE, ROWS_PER_TILE)],
                  out_hbm.at[pl.ds(c * NP + s * ROWS_PER_TILE, ROWS_PER_TILE)])


_hist = pl.kernel(
    _hist_body,
    out_type=jax.ShapeDtypeStruct((NC * NP,), jnp.float32),
    mesh=_mesh,
    scratch_types=[
        pltpu.VMEM_SHARED((NP,), jnp.float32),
        pltpu.VMEM((HCHUNKS_PER_W, 128), jnp.int32),
        pltpu.VMEM((128,), jnp.float32),
        pltpu.VMEM((128,), jnp.float32),
        pltpu.SemaphoreType.DMA,
    ],
)


def _spmm_body(n1_hbm, n2_hbm, dv_hbm, h_hbm, out_hbm, acc_sh,
               n1b0, n1b1, n2b0, n2b1, dvb0, dvb1,
               rows_v0, rows_v1,
               gsem0, gsem1, ssem0, ssem1,
               asem0, asem1, bsem0, bsem1, dsem0, dsem1):
  c = lax.axis_index("c")
  s = lax.axis_index("s")
  n_my = jnp.where(c == 0, CH0_W, CH1_W)
  t0 = jnp.where(c == 0, s * CH0_W, NS * CH0_W + s * CH1_W)
  _zero_vmem_2d(rows_v0, CH)
  for k in range(ROWS_PER_TILE // CH):
    pltpu.sync_copy(rows_v0, acc_sh.at[pl.ds(s * ROWS_PER_TILE + k * CH, CH)])

  rows = (rows_v0, rows_v1)
  n1b = (n1b0, n1b1)
  n2b = (n2b0, n2b1)
  dvb = (dvb0, dvb1)
  gsem = (gsem0, gsem1)
  ssem = (ssem0, ssem1)
  asem = (asem0, asem1)
  bsem = (bsem0, bsem1)
  dsem = (dsem0, dsem1)

  def start_n1(ch, b):
    pltpu.async_copy(n1_hbm.at[t0 + ch], n1b[b], asem[b])

  def wait_n1(b):
    pltpu.make_async_copy(n1_hbm.at[0], n1b[b], asem[b]).wait()

  def start_n2(ch, b):
    pltpu.async_copy(n2_hbm.at[t0 + ch], n2b[b], bsem[b])

  def wait_n2(b):
    pltpu.make_async_copy(n2_hbm.at[0], n2b[b], bsem[b]).wait()

  def start_dv(ch, b):
    pltpu.async_copy(dv_hbm.at[t0 + ch], dvb[b], dsem[b])

  def wait_dv(b):
    pltpu.make_async_copy(dv_hbm.at[0], dvb[b], dsem[b]).wait()

  def start_gather(b):
    pltpu.async_copy(h_hbm.at[n2b[b]], rows[b], gsem[b])

  def wait_gather(b):
    pltpu.make_async_copy(h_hbm.at[n2b[b]], rows[b], gsem[b]).wait()

  def start_scatter(b):
    pltpu.async_copy(rows[b], acc_sh.at[n1b[b]], ssem[b], add=True)

  def wait_scatter(b):
    pltpu.make_async_copy(rows[b], acc_sh.at[n1b[b]], ssem[b]).wait()

  def scale(b):
    r = rows[b]
    for gq in range(CH // 16):
      dvv = dvb[b][pl.ds(gq * 16, 16)]
      wg = jnp.exp(-(dvv * dvv))
      for t in range(16):
        w = wg[t]
        e = gq * 16 + t
        for j in range(8):
          sl = pl.ds(j * 16, 16)
          r[e, sl] = r[e, sl] * w

  nit = n_my // 2
  plsc.subcore_barrier()
  for b in range(2):
    start_n2(b, b)
    start_dv(b, b)
    start_n1(b, b)
  for b in range(2):
    wait_n2(b)
    start_gather(b)

  def body(i, carry):
    c0 = 2 * i
    more = i < nit - 1

    wait_gather(0)

    @pl.when(more)
    def _():
      start_n2(c0 + 2, 0)
    wait_dv(0)
    scale(0)

    @pl.when(more)
    def _():
      start_dv(c0 + 2, 0)
    wait_n1(0)
    start_scatter(0)

    wait_gather(1)

    @pl.when(more)
    def _():
      start_n2(c0 + 3, 1)
    wait_dv(1)
    scale(1)

    @pl.when(more)
    def _():
      start_dv(c0 + 3, 1)
    wait_n1(1)
    start_scatter(1)

    wait_scatter(0)

    @pl.when(more)
    def _():
      start_n1(c0 + 2, 0)
      wait_n2(0)
      start_gather(0)
    wait_scatter(1)

    @pl.when(more)
    def _():
      start_n1(c0 + 3, 1)
      wait_n2(1)
      start_gather(1)
    return carry
  lax.fori_loop(0, nit, body, None)

  plsc.subcore_barrier()
  for k in range(ROWS_PER_TILE // CH):
    r = s * ROWS_PER_TILE + k * CH
    pltpu.sync_copy(acc_sh.at[pl.ds(r, CH)], out_hbm.at[pl.ds(c * NP + r, CH)])


_spmm = pl.kernel(
    _spmm_body,
    out_type=jax.ShapeDtypeStruct((NC * NP, D), jnp.float32),
    mesh=_mesh,
    scratch_types=(
        [pltpu.VMEM_SHARED((NP, D), jnp.float32)]
        + [pltpu.VMEM((CH,), jnp.int32) for _ in range(4)]
        + [pltpu.VMEM((CH,), jnp.float32) for _ in range(2)]
        + [pltpu.VMEM((CH, D), jnp.float32) for _ in range(2)]
        + [pltpu.SemaphoreType.DMA for _ in range(10)]
    ),
)


def _gather_body(idx_hbm, enc_hbm, out_hbm, idx_v0, idx_v1, rows_v0, rows_v1,
                 sem0, sem1, wsem0, wsem1):
  c = lax.axis_index("c")
  s = lax.axis_index("s")
  wid = c * NS + s
  base0 = wid * (GCHUNKS_PER_W * 128)
  base1 = base0 + 128
  pltpu.sync_copy(idx_hbm.at[pl.ds(base0, 128)], idx_v0)
  pltpu.async_copy(enc_hbm.at[idx_v0], rows_v0, sem0)
  pltpu.sync_copy(idx_hbm.at[pl.ds(base1, 128)], idx_v1)
  pltpu.async_copy(enc_hbm.at[idx_v1], rows_v1, sem1)
  pltpu.make_async_copy(enc_hbm.at[idx_v0], rows_v0, sem0).wait()
  pltpu.async_copy(rows_v0, out_hbm.at[pl.ds(base0, 128)], wsem0)
  pltpu.make_async_copy(enc_hbm.at[idx_v1], rows_v1, sem1).wait()
  pltpu.async_copy(rows_v1, out_hbm.at[pl.ds(base1, 128)], wsem1)
  pltpu.make_async_copy(rows_v0, out_hbm.at[pl.ds(base0, 128)], wsem0).wait()
  pltpu.make_async_copy(rows_v1, out_hbm.at[pl.ds(base1, 128)], wsem1).wait()


_gather = pl.kernel(
    _gather_body,
    out_type=jax.ShapeDtypeStruct((GTOT, D), jnp.float32),
    mesh=_mesh,
    scratch_types=[
        pltpu.VMEM((128,), jnp.int32),
        pltpu.VMEM((128,), jnp.int32),
        pltpu.VMEM((128, D), jnp.float32),
        pltpu.VMEM((128, D), jnp.float32),
        pltpu.SemaphoreType.DMA,
        pltpu.SemaphoreType.DMA,
        pltpu.SemaphoreType.DMA,
        pltpu.SemaphoreType.DMA,
    ],
)


_RB = 512
_GRID = NP // _RB


def _mm_scale_body(x_ref, dp_ref, w_ref, b_ref, o_ref):
  dp = dp_ref[...]
  sc = lax.rsqrt(dp[0] + dp[1] + 1.0)
  h = lax.dot_general(x_ref[...], w_ref[...], (((1,), (1,)), ((), ())),
                      preferred_element_type=jnp.float32)
  h = (h + b_ref[...]) * sc[:, None]
  rows = pl.program_id(0) * _RB + lax.broadcasted_iota(jnp.int32, h.shape, 0)
  o_ref[...] = jnp.where(rows < N, h, 0.0)


def _lrelu_norm(u):
  e = jnp.where(u >= 0, u, 0.01 * u)
  nrm = jnp.sqrt(jnp.sum(e * e, axis=1, keepdims=True))
  return e / jnp.maximum(nrm, 1e-12)


def _fuse_body(p0_ref, p1_ref, dp_ref, w_ref, b_ref, o_ref):
  dp = dp_ref[...]
  sc = lax.rsqrt(dp[0] + dp[1] + 1.0)
  e = _lrelu_norm(p0_ref[...] + p1_ref[...])
  h = lax.dot_general(e, w_ref[...], (((1,), (1,)), ((), ())),
                      preferred_element_type=jnp.float32)
  h = (h + b_ref[...]) * sc[:, None]
  rows = pl.program_id(0) * _RB + lax.broadcasted_iota(jnp.int32, h.shape, 0)
  o_ref[...] = jnp.where(rows < N, h, 0.0)


def _norm_body(p0_ref, p1_ref, o_ref):
  o_ref[...] = _lrelu_norm(p0_ref[...] + p1_ref[...])


def _attn_body(x_ref, wqkv_ref, bqkv_ref, o_ref):
  x = x_ref[0]
  qkv = lax.dot_general(x, wqkv_ref[...], (((1,), (1,)), ((), ())),
                        preferred_element_type=jnp.float32) + bqkv_ref[...]
  q = qkv[:, :D]
  k = qkv[:, D:2 * D]
  v = qkv[:, 2 * D:]
  lane = lax.broadcasted_iota(jnp.int32, (1, D), 1) // DH
  scale = 1.0 / jnp.sqrt(jnp.float32(DH))
  acc = jnp.zeros((1, D), jnp.float32)
  for h in range(H):
    mh = (lane == h).astype(jnp.float32)
    sco = lax.dot_general(q * mh, k, (((1,), (1,)), ((), ())),
                          preferred_element_type=jnp.float32) * scale
    sco = sco - jnp.max(sco, axis=1, keepdims=True)
    ex = jnp.exp(sco)
    p = ex / jnp.sum(ex, axis=1, keepdims=True)
    m = jnp.mean(p, axis=0, keepdims=True)
    acc = acc + lax.dot_general(m, v * mh, (((1,), (0,)), ((), ())),
                                preferred_element_type=jnp.float32)
  o_ref[0] = acc


def _proj_body(x_ref, w_ref, b_ref, o_ref):
  o_ref[...] = lax.dot_general(x_ref[...], w_ref[...], (((1,), (1,)), ((), ())),
                               preferred_element_type=jnp.float32) + b_ref[...]


def _full(shape):
  return pl.BlockSpec(shape, lambda b: tuple(0 for _ in shape))


_mm_scale = pl.pallas_call(
    _mm_scale_body,
    grid=(_GRID,),
    in_specs=[
        pl.BlockSpec((_RB, D), lambda b: (b, 0)),
        pl.BlockSpec((2, _RB), lambda b: (0, b)),
        _full((D, D)),
        _full((1, D)),
    ],
    out_specs=pl.BlockSpec((_RB, D), lambda b: (b, 0)),
    out_shape=jax.ShapeDtypeStruct((NP, D), jnp.float32),
)

_fuse = pl.pallas_call(
    _fuse_body,
    grid=(_GRID,),
    in_specs=[
        pl.BlockSpec((_RB, D), lambda b: (b, 0)),
        pl.BlockSpec((_RB, D), lambda b: (b + _GRID, 0)),
        pl.BlockSpec((2, _RB), lambda b: (0, b)),
        _full((D, D)),
        _full((1, D)),
    ],
    out_specs=pl.BlockSpec((_RB, D), lambda b: (b, 0)),
    out_shape=jax.ShapeDtypeStruct((NP, D), jnp.float32),
)

_norm = pl.pallas_call(
    _norm_body,
    grid=(_GRID,),
    in_specs=[
        pl.BlockSpec((_RB, D), lambda b: (b, 0)),
        pl.BlockSpec((_RB, D), lambda b: (b + _GRID, 0)),
    ],
    out_specs=pl.BlockSpec((_RB, D), lambda b: (b, 0)),
    out_shape=jax.ShapeDtypeStruct((NP, D), jnp.float32),
)

_attn = pl.pallas_call(
    _attn_body,
    grid=(B,),
    in_specs=[
        pl.BlockSpec((1, L, D), lambda b: (b, 0, 0)),
        _full((3 * D, D)),
        _full((1, 3 * D)),
    ],
    out_specs=pl.BlockSpec((1, 1, D), lambda b: (b, 0, 0)),
    out_shape=jax.ShapeDtypeStruct((B, 1, D), jnp.float32),
)

_proj = pl.pallas_call(
    _proj_body,
    grid=(1,),
    in_specs=[_full((B, D)), _full((D, D)), _full((1, D))],
    out_specs=_full((B, D)),
    out_shape=jax.ShapeDtypeStruct((B, D), jnp.float32),
)


def kernel(poi_table, dist_edges, dist_vec, data_x, data_poi,
           W0, b0, W1, b1, Wqkv, bqkv, Wo, bo):
  i32 = jnp.int32
  f32 = jnp.float32
  d0 = dist_edges[0].astype(i32)
  d1 = dist_edges[1].astype(i32)
  loop = jnp.arange(N, dtype=i32)
  epad = ETOT - (2 * E + N)
  pad_idx = jnp.full((epad,), NP - 1, i32)
  n1 = jnp.concatenate([d0, d1, loop, pad_idx])
  n2 = jnp.concatenate([d1, d0, loop, pad_idx])
  dv = jnp.concatenate([dist_vec.astype(f32), dist_vec.astype(f32),
                        jnp.zeros((N,), f32), jnp.full((epad,), 1e9, f32)])
  n1c = n1.reshape(ECHUNKS, CH)
  n2c = n2.reshape(ECHUNKS, CH)
  dv2 = dv.reshape(ECHUNKS, CH)
  nodes = jnp.concatenate(
      [d0, d1, jnp.full((HTOT - 2 * E,), NP - 1, i32)]).reshape(HTOT // 128, 128)

  degp = _hist(nodes).reshape(NC, NP)
  ptab = jnp.pad(poi_table.astype(f32), ((0, NP - N), (0, 0)))
  b0r = b0.reshape(1, D).astype(f32)
  b1r = b1.reshape(1, D).astype(f32)

  h1 = _mm_scale(ptab, degp, W0.astype(f32), b0r)
  p1 = _spmm(n1c, n2c, dv2, h1)
  h2 = _fuse(p1, p1, degp, W1.astype(f32), b1r)
  p2 = _spmm(n1c, n2c, dv2, h2)
  enc = _norm(p2, p2)

  gidx = jnp.concatenate([data_x.astype(i32), data_poi.astype(i32),
                          jnp.zeros((GTOT - B * L - B,), i32)])
  g = _gather(gidx, enc)
  poi_embed = g[B * L:B * L + B]
  seq3 = g[:B * L].reshape(B, L, D)
  ctxm = _attn(seq3, Wqkv.astype(f32),
               bqkv.reshape(1, 3 * D).astype(f32)).reshape(B, D)
  aggr = _proj(ctxm, Wo.astype(f32), bo.reshape(1, D).astype(f32))
  return (aggr, poi_embed)


# speedup vs baseline: 1.0995x; 1.0191x over previous
import functools

import jax
import jax.numpy as jnp
from jax import lax
from jax.experimental import pallas as pl
from jax.experimental.pallas import tpu as pltpu
from jax.experimental.pallas import tpu_sc as plsc

N = 10000
NP = 10240
E = 320000
D = 128
H = 16
DH = 8
B = 128
L = 50

NC = 2
NS = 16
NW = NC * NS

CH = 128
NCH_W = 160
CH0_W = 184
CH1_W = 2 * NCH_W - CH0_W
ETOT = NW * NCH_W * CH
ECHUNKS = ETOT // CH

HTOT = 655360
HCHUNKS_PER_W = HTOT // 128 // NW

GTOT = 8192
GCHUNKS_PER_W = GTOT // 128 // NW

ROWS_PER_TILE = NP // NS

_mesh = plsc.VectorSubcoreMesh(
    core_axis_name="c", subcore_axis_name="s", num_cores=NC, num_subcores=NS)


def _zero_vmem_2d(ref, nrows):
  def row(r, _):
    for j in range(8):
      ref[r, pl.ds(j * 16, 16)] = jnp.zeros((16,), jnp.float32)
    return _
  lax.fori_loop(0, nrows, row, None)


def _hist_body(nodes_hbm, out_hbm, deg_sh, idx_all, ones_v, zbuf_v, sem):
  c = lax.axis_index("c")
  s = lax.axis_index("s")
  wid = c * NS + s
  pltpu.sync_copy(nodes_hbm.at[pl.ds(wid * HCHUNKS_PER_W, HCHUNKS_PER_W)],
                  idx_all)
  for j in range(8):
    ones_v[pl.ds(j * 16, 16)] = jnp.ones((16,), jnp.float32)
    zbuf_v[pl.ds(j * 16, 16)] = jnp.zeros((16,), jnp.float32)
  for k in range(ROWS_PER_TILE // 128):
    pltpu.sync_copy(zbuf_v, deg_sh.at[pl.ds(s * ROWS_PER_TILE + k * 128, 128)])
  plsc.subcore_barrier()

  def group(g, carry):
    for k in range(8):
      pltpu.async_copy(ones_v, deg_sh.at[idx_all.at[g * 8 + k]], sem, add=True)
    for k in range(8):
      pltpu.make_async_copy(ones_v, deg_sh.at[idx_all.at[g * 8 + k]],
                            sem).wait()
    return carry
  lax.fori_loop(0, HCHUNKS_PER_W // 8, group, None)
  plsc.subcore_barrier()
  pltpu.sync_copy(deg_sh.at[pl.ds(s * ROWS_PER_TILE, ROWS_PER_TILE)],
                  out_hbm.at[pl.ds(c * NP + s * ROWS_PER_TILE, ROWS_PER_TILE)])


_hist = pl.kernel(
    _hist_body,
    out_type=jax.ShapeDtypeStruct((NC * NP,), jnp.float32),
    mesh=_mesh,
    scratch_types=[
        pltpu.VMEM_SHARED((NP,), jnp.float32),
        pltpu.VMEM((HCHUNKS_PER_W, 128), jnp.int32),
        pltpu.VMEM((128,), jnp.float32),
        pltpu.VMEM((128,), jnp.float32),
        pltpu.SemaphoreType.DMA,
    ],
)


def _spmm_body(n1_hbm, n2_hbm, dv_hbm, h_hbm, out_hbm, acc_sh,
               n1b0, n1b1, n2b0, n2b1, dvb0, dvb1,
               rows_v0, rows_v1,
               gsem0, gsem1, ssem0, ssem1,
               asem0, asem1, bsem0, bsem1, dsem0, dsem1):
  c = lax.axis_index("c")
  s = lax.axis_index("s")
  n_my = jnp.where(c == 0, CH0_W, CH1_W)
  t0 = jnp.where(c == 0, s * CH0_W, NS * CH0_W + s * CH1_W)
  _zero_vmem_2d(rows_v0, CH)
  for k in range(ROWS_PER_TILE // CH):
    pltpu.sync_copy(rows_v0, acc_sh.at[pl.ds(s * ROWS_PER_TILE + k * CH, CH)])

  rows = (rows_v0, rows_v1)
  n1b = (n1b0, n1b1)
  n2b = (n2b0, n2b1)
  dvb = (dvb0, dvb1)
  gsem = (gsem0, gsem1)
  ssem = (ssem0, ssem1)
  asem = (asem0, asem1)
  bsem = (bsem0, bsem1)
  dsem = (dsem0, dsem1)

  def start_n1(ch, b):
    pltpu.async_copy(n1_hbm.at[t0 + ch], n1b[b], asem[b])

  def wait_n1(b):
    pltpu.make_async_copy(n1_hbm.at[0], n1b[b], asem[b]).wait()

  def start_n2(ch, b):
    pltpu.async_copy(n2_hbm.at[t0 + ch], n2b[b], bsem[b])

  def wait_n2(b):
    pltpu.make_async_copy(n2_hbm.at[0], n2b[b], bsem[b]).wait()

  def start_dv(ch, b):
    pltpu.async_copy(dv_hbm.at[t0 + ch], dvb[b], dsem[b])

  def wait_dv(b):
    pltpu.make_async_copy(dv_hbm.at[0], dvb[b], dsem[b]).wait()

  def start_gather(b):
    pltpu.async_copy(h_hbm.at[n2b[b]], rows[b], gsem[b])

  def wait_gather(b):
    pltpu.make_async_copy(h_hbm.at[n2b[b]], rows[b], gsem[b]).wait()

  def start_scatter(b):
    pltpu.async_copy(rows[b], acc_sh.at[n1b[b]], ssem[b], add=True)

  def wait_scatter(b):
    pltpu.make_async_copy(rows[b], acc_sh.at[n1b[b]], ssem[b]).wait()

  def scale(b):
    r = rows[b]
    for gq in range(CH // 16):
      dvv = dvb[b][pl.ds(gq * 16, 16)]
      wg = jnp.exp(-(dvv * dvv))
      for t in range(16):
        w = wg[t]
        e = gq * 16 + t
        for j in range(8):
          sl = pl.ds(j * 16, 16)
          r[e, sl] = r[e, sl] * w

  nit = n_my // 2
  plsc.subcore_barrier()
  for b in range(2):
    start_n2(b, b)
    start_dv(b, b)
    start_n1(b, b)
  for b in range(2):
    wait_n2(b)
    start_gather(b)

  def body(i, carry):
    c0 = 2 * i
    more = i < nit - 1

    wait_gather(0)

    @pl.when(more)
    def _():
      start_n2(c0 + 2, 0)
    wait_dv(0)
    scale(0)

    @pl.when(more)
    def _():
      start_dv(c0 + 2, 0)
    wait_n1(0)
    start_scatter(0)

    wait_gather(1)

    @pl.when(more)
    def _():
      start_n2(c0 + 3, 1)
    wait_dv(1)
    scale(1)

    @pl.when(more)
    def _():
      start_dv(c0 + 3, 1)
    wait_n1(1)
    start_scatter(1)

    wait_scatter(0)

    @pl.when(more)
    def _():
      start_n1(c0 + 2, 0)
      wait_n2(0)
      start_gather(0)
    wait_scatter(1)

    @pl.when(more)
    def _():
      start_n1(c0 + 3, 1)
      wait_n2(1)
      start_gather(1)
    return carry
  lax.fori_loop(0, nit, body, None)

  plsc.subcore_barrier()
  for k in range(ROWS_PER_TILE // CH):
    r = s * ROWS_PER_TILE + k * CH
    pltpu.sync_copy(acc_sh.at[pl.ds(r, CH)], out_hbm.at[pl.ds(c * NP + r, CH)])


_spmm = pl.kernel(
    _spmm_body,
    out_type=jax.ShapeDtypeStruct((NC * NP, D), jnp.float32),
    mesh=_mesh,
    scratch_types=(
        [pltpu.VMEM_SHARED((NP, D), jnp.float32)]
        + [pltpu.VMEM((CH,), jnp.int32) for _ in range(4)]
        + [pltpu.VMEM((CH,), jnp.float32) for _ in range(2)]
        + [pltpu.VMEM((CH, D), jnp.float32) for _ in range(2)]
        + [pltpu.SemaphoreType.DMA for _ in range(10)]
    ),
)


def _gather_body(idx_hbm, enc_hbm, out_hbm, idx_v0, idx_v1, rows_v0, rows_v1,
                 sem0, sem1, wsem0, wsem1):
  c = lax.axis_index("c")
  s = lax.axis_index("s")
  wid = c * NS + s
  base0 = wid * (GCHUNKS_PER_W * 128)
  base1 = base0 + 128
  pltpu.sync_copy(idx_hbm.at[pl.ds(base0, 128)], idx_v0)
  pltpu.async_copy(enc_hbm.at[idx_v0], rows_v0, sem0)
  pltpu.sync_copy(idx_hbm.at[pl.ds(base1, 128)], idx_v1)
  pltpu.async_copy(enc_hbm.at[idx_v1], rows_v1, sem1)
  pltpu.make_async_copy(enc_hbm.at[idx_v0], rows_v0, sem0).wait()
  pltpu.async_copy(rows_v0, out_hbm.at[pl.ds(base0, 128)], wsem0)
  pltpu.make_async_copy(enc_hbm.at[idx_v1], rows_v1, sem1).wait()
  pltpu.async_copy(rows_v1, out_hbm.at[pl.ds(base1, 128)], wsem1)
  pltpu.make_async_copy(rows_v0, out_hbm.at[pl.ds(base0, 128)], wsem0).wait()
  pltpu.make_async_copy(rows_v1, out_hbm.at[pl.ds(base1, 128)], wsem1).wait()


_gather = pl.kernel(
    _gather_body,
    out_type=jax.ShapeDtypeStruct((GTOT, D), jnp.float32),
    mesh=_mesh,
    scratch_types=[
        pltpu.VMEM((128,), jnp.int32),
        pltpu.VMEM((128,), jnp.int32),
        pltpu.VMEM((128, D), jnp.float32),
        pltpu.VMEM((128, D), jnp.float32),
        pltpu.SemaphoreType.DMA,
        pltpu.SemaphoreType.DMA,
        pltpu.SemaphoreType.DMA,
        pltpu.SemaphoreType.DMA,
    ],
)


_RB = 512
_GRID = NP // _RB


def _mm_scale_body(x_ref, dp_ref, w_ref, b_ref, o_ref):
  dp = dp_ref[...]
  sc = lax.rsqrt(dp[0] + dp[1] + 1.0)
  h = lax.dot_general(x_ref[...], w_ref[...], (((1,), (1,)), ((), ())),
                      preferred_element_type=jnp.float32)
  h = (h + b_ref[...]) * sc[:, None]
  rows = pl.program_id(0) * _RB + lax.broadcasted_iota(jnp.int32, h.shape, 0)
  o_ref[...] = jnp.where(rows < N, h, 0.0)


def _lrelu_norm(u):
  e = jnp.where(u >= 0, u, 0.01 * u)
  nrm = jnp.sqrt(jnp.sum(e * e, axis=1, keepdims=True))
  return e / jnp.maximum(nrm, 1e-12)


def _fuse_body(p0_ref, p1_ref, dp_ref, w_ref, b_ref, o_ref):
  dp = dp_ref[...]
  sc = lax.rsqrt(dp[0] + dp[1] + 1.0)
  e = _lrelu_norm(p0_ref[...] + p1_ref[...])
  h = lax.dot_general(e, w_ref[...], (((1,), (1,)), ((), ())),
                      preferred_element_type=jnp.float32)
  h = (h + b_ref[...]) * sc[:, None]
  rows = pl.program_id(0) * _RB + lax.broadcasted_iota(jnp.int32, h.shape, 0)
  o_ref[...] = jnp.where(rows < N, h, 0.0)


def _norm_body(p0_ref, p1_ref, o_ref):
  o_ref[...] = _lrelu_norm(p0_ref[...] + p1_ref[...])


def _attn_body(x_ref, wqkv_ref, bqkv_ref, o_ref):
  x = x_ref[0]
  qkv = lax.dot_general(x, wqkv_ref[...], (((1,), (1,)), ((), ())),
                        preferred_element_type=jnp.float32) + bqkv_ref[...]
  q = qkv[:, :D]
  k = qkv[:, D:2 * D]
  v = qkv[:, 2 * D:]
  lane = lax.broadcasted_iota(jnp.int32, (1, D), 1) // DH
  scale = 1.0 / jnp.sqrt(jnp.float32(DH))
  acc = jnp.zeros((1, D), jnp.float32)
  for h in range(H):
    mh = (lane == h).astype(jnp.float32)
    sco = lax.dot_general(q * mh, k, (((1,), (1,)), ((), ())),
                          preferred_element_type=jnp.float32) * scale
    sco = sco - jnp.max(sco, axis=1, keepdims=True)
    ex = jnp.exp(sco)
    p = ex / jnp.sum(ex, axis=1, keepdims=True)
    m = jnp.mean(p, axis=0, keepdims=True)
    acc = acc + lax.dot_general(m, v * mh, (((1,), (0,)), ((), ())),
                                preferred_element_type=jnp.float32)
  o_ref[0] = acc


def _proj_body(x_ref, w_ref, b_ref, o_ref):
  o_ref[...] = lax.dot_general(x_ref[...], w_ref[...], (((1,), (1,)), ((), ())),
                               preferred_element_type=jnp.float32) + b_ref[...]


def _full(shape):
  return pl.BlockSpec(shape, lambda b: tuple(0 for _ in shape))


_mm_scale = pl.pallas_call(
    _mm_scale_body,
    grid=(_GRID,),
    in_specs=[
        pl.BlockSpec((_RB, D), lambda b: (b, 0)),
        pl.BlockSpec((2, _RB), lambda b: (0, b)),
        _full((D, D)),
        _full((1, D)),
    ],
    out_specs=pl.BlockSpec((_RB, D), lambda b: (b, 0)),
    out_shape=jax.ShapeDtypeStruct((NP, D), jnp.float32),
)

_fuse = pl.pallas_call(
    _fuse_body,
    grid=(_GRID,),
    in_specs=[
        pl.BlockSpec((_RB, D), lambda b: (b, 0)),
        pl.BlockSpec((_RB, D), lambda b: (b + _GRID, 0)),
        pl.BlockSpec((2, _RB), lambda b: (0, b)),
        _full((D, D)),
        _full((1, D)),
    ],
    out_specs=pl.BlockSpec((_RB, D), lambda b: (b, 0)),
    out_shape=jax.ShapeDtypeStruct((NP, D), jnp.float32),
)

_norm = pl.pallas_call(
    _norm_body,
    grid=(_GRID,),
    in_specs=[
        pl.BlockSpec((_RB, D), lambda b: (b, 0)),
        pl.BlockSpec((_RB, D), lambda b: (b + _GRID, 0)),
    ],
    out_specs=pl.BlockSpec((_RB, D), lambda b: (b, 0)),
    out_shape=jax.ShapeDtypeStruct((NP, D), jnp.float32),
)

_attn = pl.pallas_call(
    _attn_body,
    grid=(B,),
    in_specs=[
        pl.BlockSpec((1, L, D), lambda b: (b, 0, 0)),
        _full((3 * D, D)),
        _full((1, 3 * D)),
    ],
    out_specs=pl.BlockSpec((1, 1, D), lambda b: (b, 0, 0)),
    out_shape=jax.ShapeDtypeStruct((B, 1, D), jnp.float32),
)

_proj = pl.pallas_call(
    _proj_body,
    grid=(1,),
    in_specs=[_full((B, D)), _full((D, D)), _full((1, D))],
    out_specs=_full((B, D)),
    out_shape=jax.ShapeDtypeStruct((B, D), jnp.float32),
)


def kernel(poi_table, dist_edges, dist_vec, data_x, data_poi,
           W0, b0, W1, b1, Wqkv, bqkv, Wo, bo):
  i32 = jnp.int32
  f32 = jnp.float32
  d0 = dist_edges[0].astype(i32)
  d1 = dist_edges[1].astype(i32)
  loop = jnp.arange(N, dtype=i32)
  epad = ETOT - (2 * E + N)
  pad_idx = jnp.full((epad,), NP - 1, i32)
  n1 = jnp.concatenate([d0, d1, loop, pad_idx])
  n2 = jnp.concatenate([d1, d0, loop, pad_idx])
  dv = jnp.concatenate([dist_vec.astype(f32), dist_vec.astype(f32),
                        jnp.zeros((N,), f32), jnp.full((epad,), 1e9, f32)])
  n1c = n1.reshape(ECHUNKS, CH)
  n2c = n2.reshape(ECHUNKS, CH)
  dv2 = dv.reshape(ECHUNKS, CH)
  nodes = jnp.concatenate(
      [d0, d1, jnp.full((HTOT - 2 * E,), NP - 1, i32)]).reshape(HTOT // 128, 128)

  degp = _hist(nodes).reshape(NC, NP)
  ptab = jnp.pad(poi_table.astype(f32), ((0, NP - N), (0, 0)))
  b0r = b0.reshape(1, D).astype(f32)
  b1r = b1.reshape(1, D).astype(f32)

  h1 = _mm_scale(ptab, degp, W0.astype(f32), b0r)
  p1 = _spmm(n1c, n2c, dv2, h1)
  h2 = _fuse(p1, p1, degp, W1.astype(f32), b1r)
  p2 = _spmm(n1c, n2c, dv2, h2)
  enc = _norm(p2, p2)

  gidx = jnp.concatenate([data_x.astype(i32), data_poi.astype(i32),
                          jnp.zeros((GTOT - B * L - B,), i32)])
  g = _gather(gidx, enc)
  poi_embed = g[B * L:B * L + B]
  seq3 = g[:B * L].reshape(B, L, D)
  ctxm = _attn(seq3, Wqkv.astype(f32),
               bqkv.reshape(1, 3 * D).astype(f32)).reshape(B, D)
  aggr = _proj(ctxm, Wo.astype(f32), bo.reshape(1, D).astype(f32))
  return (aggr, poi_embed)


# speedup vs baseline: 1.1030x; 1.0032x over previous
import functools

import jax
import jax.numpy as jnp
from jax import lax
from jax.experimental import pallas as pl
from jax.experimental.pallas import tpu as pltpu
from jax.experimental.pallas import tpu_sc as plsc

N = 10000
NP = 10240
E = 320000
D = 128
H = 16
DH = 8
B = 128
L = 50

NC = 2
NS = 16
NW = NC * NS

CH = 128
NCH_W = 160
CH0_W = 192
CH1_W = 2 * NCH_W - CH0_W
ETOT = NW * NCH_W * CH
ECHUNKS = ETOT // CH

HTOT = 655360
HCHUNKS_PER_W = HTOT // 128 // NW

GTOT = 8192
GCHUNKS_PER_W = GTOT // 128 // NW

ROWS_PER_TILE = NP // NS

_mesh = plsc.VectorSubcoreMesh(
    core_axis_name="c", subcore_axis_name="s", num_cores=NC, num_subcores=NS)


def _zero_vmem_2d(ref, nrows):
  def row(r, _):
    for j in range(8):
      ref[r, pl.ds(j * 16, 16)] = jnp.zeros((16,), jnp.float32)
    return _
  lax.fori_loop(0, nrows, row, None)


def _hist_body(nodes_hbm, out_hbm, deg_sh, idx_all, ones_v, zbuf_v, sem):
  c = lax.axis_index("c")
  s = lax.axis_index("s")
  wid = c * NS + s
  pltpu.sync_copy(nodes_hbm.at[pl.ds(wid * HCHUNKS_PER_W, HCHUNKS_PER_W)],
                  idx_all)
  for j in range(8):
    ones_v[pl.ds(j * 16, 16)] = jnp.ones((16,), jnp.float32)
    zbuf_v[pl.ds(j * 16, 16)] = jnp.zeros((16,), jnp.float32)
  for k in range(ROWS_PER_TILE // 128):
    pltpu.sync_copy(zbuf_v, deg_sh.at[pl.ds(s * ROWS_PER_TILE + k * 128, 128)])
  plsc.subcore_barrier()

  def group(g, carry):
    for k in range(8):
      pltpu.async_copy(ones_v, deg_sh.at[idx_all.at[g * 8 + k]], sem, add=True)
    for k in range(8):
      pltpu.make_async_copy(ones_v, deg_sh.at[idx_all.at[g * 8 + k]],
                            sem).wait()
    return carry
  lax.fori_loop(0, HCHUNKS_PER_W // 8, group, None)
  plsc.subcore_barrier()
  pltpu.sync_copy(deg_sh.at[pl.ds(s * ROWS_PER_TILE, ROWS_PER_TILE)],
                  out_hbm.at[pl.ds(c * NP + s * ROWS_PER_TILE, ROWS_PER_TILE)])


_hist = pl.kernel(
    _hist_body,
    out_type=jax.ShapeDtypeStruct((NC * NP,), jnp.float32),
    mesh=_mesh,
    scratch_types=[
        pltpu.VMEM_SHARED((NP,), jnp.float32),
        pltpu.VMEM((HCHUNKS_PER_W, 128), jnp.int32),
        pltpu.VMEM((128,), jnp.float32),
        pltpu.VMEM((128,), jnp.float32),
        pltpu.SemaphoreType.DMA,
    ],
)


def _spmm_body(n1_hbm, n2_hbm, dv_hbm, h_hbm, out_hbm, acc_sh,
               n1b0, n1b1, n2b0, n2b1, dvb0, dvb1,
               rows_v0, rows_v1,
               gsem0, gsem1, ssem0, ssem1,
               asem0, asem1, bsem0, bsem1, dsem0, dsem1):
  c = lax.axis_index("c")
  s = lax.axis_index("s")
  n_my = jnp.where(c == 0, CH0_W, CH1_W)
  t0 = jnp.where(c == 0, s * CH0_W, NS * CH0_W + s * CH1_W)
  _zero_vmem_2d(rows_v0, CH)
  for k in range(ROWS_PER_TILE // CH):
    pltpu.sync_copy(rows_v0, acc_sh.at[pl.ds(s * ROWS_PER_TILE + k * CH, CH)])

  rows = (rows_v0, rows_v1)
  n1b = (n1b0, n1b1)
  n2b = (n2b0, n2b1)
  dvb = (dvb0, dvb1)
  gsem = (gsem0, gsem1)
  ssem = (ssem0, ssem1)
  asem = (asem0, asem1)
  bsem = (bsem0, bsem1)
  dsem = (dsem0, dsem1)

  def start_n1(ch, b):
    pltpu.async_copy(n1_hbm.at[t0 + ch], n1b[b], asem[b])

  def wait_n1(b):
    pltpu.make_async_copy(n1_hbm.at[0], n1b[b], asem[b]).wait()

  def start_n2(ch, b):
    pltpu.async_copy(n2_hbm.at[t0 + ch], n2b[b], bsem[b])

  def wait_n2(b):
    pltpu.make_async_copy(n2_hbm.at[0], n2b[b], bsem[b]).wait()

  def start_dv(ch, b):
    pltpu.async_copy(dv_hbm.at[t0 + ch], dvb[b], dsem[b])

  def wait_dv(b):
    pltpu.make_async_copy(dv_hbm.at[0], dvb[b], dsem[b]).wait()

  def start_gather(b):
    pltpu.async_copy(h_hbm.at[n2b[b]], rows[b], gsem[b])

  def wait_gather(b):
    pltpu.make_async_copy(h_hbm.at[n2b[b]], rows[b], gsem[b]).wait()

  def start_scatter(b):
    pltpu.async_copy(rows[b], acc_sh.at[n1b[b]], ssem[b], add=True)

  def wait_scatter(b):
    pltpu.make_async_copy(rows[b], acc_sh.at[n1b[b]], ssem[b]).wait()

  def scale(b):
    r = rows[b]
    for gq in range(CH // 16):
      dvv = dvb[b][pl.ds(gq * 16, 16)]
      wg = jnp.exp(-(dvv * dvv))
      for t in range(16):
        w = wg[t]
        e = gq * 16 + t
        for j in range(8):
          sl = pl.ds(j * 16, 16)
          r[e, sl] = r[e, sl] * w

  nit = n_my // 2
  plsc.subcore_barrier()
  for b in range(2):
    start_n2(b, b)
    start_dv(b, b)
    start_n1(b, b)
  for b in range(2):
    wait_n2(b)
    start_gather(b)

  def body(i, carry):
    c0 = 2 * i
    more = i < nit - 1

    wait_gather(0)

    @pl.when(more)
    def _():
      start_n2(c0 + 2, 0)
    wait_dv(0)
    scale(0)

    @pl.when(more)
    def _():
      start_dv(c0 + 2, 0)
    wait_n1(0)
    start_scatter(0)

    wait_gather(1)

    @pl.when(more)
    def _():
      start_n2(c0 + 3, 1)
    wait_dv(1)
    scale(1)

    @pl.when(more)
    def _():
      start_dv(c0 + 3, 1)
    wait_n1(1)
    start_scatter(1)

    wait_scatter(0)

    @pl.when(more)
    def _():
      start_n1(c0 + 2, 0)
      wait_n2(0)
      start_gather(0)
    wait_scatter(1)

    @pl.when(more)
    def _():
      start_n1(c0 + 3, 1)
      wait_n2(1)
      start_gather(1)
    return carry
  lax.fori_loop(0, nit, body, None)

  plsc.subcore_barrier()
  for k in range(ROWS_PER_TILE // CH):
    r = s * ROWS_PER_TILE + k * CH
    pltpu.sync_copy(acc_sh.at[pl.ds(r, CH)], out_hbm.at[pl.ds(c * NP + r, CH)])


_spmm = pl.kernel(
    _spmm_body,
    out_type=jax.ShapeDtypeStruct((NC * NP, D), jnp.float32),
    mesh=_mesh,
    scratch_types=(
        [pltpu.VMEM_SHARED((NP, D), jnp.float32)]
        + [pltpu.VMEM((CH,), jnp.int32) for _ in range(4)]
        + [pltpu.VMEM((CH,), jnp.float32) for _ in range(2)]
        + [pltpu.VMEM((CH, D), jnp.float32) for _ in range(2)]
        + [pltpu.SemaphoreType.DMA for _ in range(10)]
    ),
)


def _gather_body(idx_hbm, enc_hbm, out_hbm, idx_v0, idx_v1, rows_v0, rows_v1,
                 sem0, sem1, wsem0, wsem1):
  c = lax.axis_index("c")
  s = lax.axis_index("s")
  wid = c * NS + s
  base0 = wid * (GCHUNKS_PER_W * 128)
  base1 = base0 + 128
  pltpu.sync_copy(idx_hbm.at[pl.ds(base0, 128)], idx_v0)
  pltpu.async_copy(enc_hbm.at[idx_v0], rows_v0, sem0)
  pltpu.sync_copy(idx_hbm.at[pl.ds(base1, 128)], idx_v1)
  pltpu.async_copy(enc_hbm.at[idx_v1], rows_v1, sem1)
  pltpu.make_async_copy(enc_hbm.at[idx_v0], rows_v0, sem0).wait()
  pltpu.async_copy(rows_v0, out_hbm.at[pl.ds(base0, 128)], wsem0)
  pltpu.make_async_copy(enc_hbm.at[idx_v1], rows_v1, sem1).wait()
  pltpu.async_copy(rows_v1, out_hbm.at[pl.ds(base1, 128)], wsem1)
  pltpu.make_async_copy(rows_v0, out_hbm.at[pl.ds(base0, 128)], wsem0).wait()
  pltpu.make_async_copy(rows_v1, out_hbm.at[pl.ds(base1, 128)], wsem1).wait()


_gather = pl.kernel(
    _gather_body,
    out_type=jax.ShapeDtypeStruct((GTOT, D), jnp.float32),
    mesh=_mesh,
    scratch_types=[
        pltpu.VMEM((128,), jnp.int32),
        pltpu.VMEM((128,), jnp.int32),
        pltpu.VMEM((128, D), jnp.float32),
        pltpu.VMEM((128, D), jnp.float32),
        pltpu.SemaphoreType.DMA,
        pltpu.SemaphoreType.DMA,
        pltpu.SemaphoreType.DMA,
        pltpu.SemaphoreType.DMA,
    ],
)


_RB = 512
_GRID = NP // _RB


def _mm_scale_body(x_ref, dp_ref, w_ref, b_ref, o_ref):
  dp = dp_ref[...]
  sc = lax.rsqrt(dp[0] + dp[1] + 1.0)
  h = lax.dot_general(x_ref[...], w_ref[...], (((1,), (1,)), ((), ())),
                      preferred_element_type=jnp.float32)
  h = (h + b_ref[...]) * sc[:, None]
  rows = pl.program_id(0) * _RB + lax.broadcasted_iota(jnp.int32, h.shape, 0)
  o_ref[...] = jnp.where(rows < N, h, 0.0)


def _lrelu_norm(u):
  e = jnp.where(u >= 0, u, 0.01 * u)
  nrm = jnp.sqrt(jnp.sum(e * e, axis=1, keepdims=True))
  return e / jnp.maximum(nrm, 1e-12)


def _fuse_body(p0_ref, p1_ref, dp_ref, w_ref, b_ref, o_ref):
  dp = dp_ref[...]
  sc = lax.rsqrt(dp[0] + dp[1] + 1.0)
  e = _lrelu_norm(p0_ref[...] + p1_ref[...])
  h = lax.dot_general(e, w_ref[...], (((1,), (1,)), ((), ())),
                      preferred_element_type=jnp.float32)
  h = (h + b_ref[...]) * sc[:, None]
  rows = pl.program_id(0) * _RB + lax.broadcasted_iota(jnp.int32, h.shape, 0)
  o_ref[...] = jnp.where(rows < N, h, 0.0)


def _norm_body(p0_ref, p1_ref, o_ref):
  o_ref[...] = _lrelu_norm(p0_ref[...] + p1_ref[...])


def _attn_body(x_ref, wqkv_ref, bqkv_ref, o_ref):
  x = x_ref[0]
  qkv = lax.dot_general(x, wqkv_ref[...], (((1,), (1,)), ((), ())),
                        preferred_element_type=jnp.float32) + bqkv_ref[...]
  q = qkv[:, :D]
  k = qkv[:, D:2 * D]
  v = qkv[:, 2 * D:]
  lane = lax.broadcasted_iota(jnp.int32, (1, D), 1) // DH
  scale = 1.0 / jnp.sqrt(jnp.float32(DH))
  acc = jnp.zeros((1, D), jnp.float32)
  for h in range(H):
    mh = (lane == h).astype(jnp.float32)
    sco = lax.dot_general(q * mh, k, (((1,), (1,)), ((), ())),
                          preferred_element_type=jnp.float32) * scale
    sco = sco - jnp.max(sco, axis=1, keepdims=True)
    ex = jnp.exp(sco)
    p = ex / jnp.sum(ex, axis=1, keepdims=True)
    m = jnp.mean(p, axis=0, keepdims=True)
    acc = acc + lax.dot_general(m, v * mh, (((1,), (0,)), ((), ())),
                                preferred_element_type=jnp.float32)
  o_ref[0] = acc


def _proj_body(x_ref, w_ref, b_ref, o_ref):
  o_ref[...] = lax.dot_general(x_ref[...], w_ref[...], (((1,), (1,)), ((), ())),
                               preferred_element_type=jnp.float32) + b_ref[...]


def _full(shape):
  return pl.BlockSpec(shape, lambda b: tuple(0 for _ in shape))


_mm_scale = pl.pallas_call(
    _mm_scale_body,
    grid=(_GRID,),
    in_specs=[
        pl.BlockSpec((_RB, D), lambda b: (b, 0)),
        pl.BlockSpec((2, _RB), lambda b: (0, b)),
        _full((D, D)),
        _full((1, D)),
    ],
    out_specs=pl.BlockSpec((_RB, D), lambda b: (b, 0)),
    out_shape=jax.ShapeDtypeStruct((NP, D), jnp.float32),
)

_fuse = pl.pallas_call(
    _fuse_body,
    grid=(_GRID,),
    in_specs=[
        pl.BlockSpec((_RB, D), lambda b: (b, 0)),
        pl.BlockSpec((_RB, D), lambda b: (b + _GRID, 0)),
        pl.BlockSpec((2, _RB), lambda b: (0, b)),
        _full((D, D)),
        _full((1, D)),
    ],
    out_specs=pl.BlockSpec((_RB, D), lambda b: (b, 0)),
    out_shape=jax.ShapeDtypeStruct((NP, D), jnp.float32),
)

_norm = pl.pallas_call(
    _norm_body,
    grid=(_GRID,),
    in_specs=[
        pl.BlockSpec((_RB, D), lambda b: (b, 0)),
        pl.BlockSpec((_RB, D), lambda b: (b + _GRID, 0)),
    ],
    out_specs=pl.BlockSpec((_RB, D), lambda b: (b, 0)),
    out_shape=jax.ShapeDtypeStruct((NP, D), jnp.float32),
)

_attn = pl.pallas_call(
    _attn_body,
    grid=(B,),
    in_specs=[
        pl.BlockSpec((1, L, D), lambda b: (b, 0, 0)),
        _full((3 * D, D)),
        _full((1, 3 * D)),
    ],
    out_specs=pl.BlockSpec((1, 1, D), lambda b: (b, 0, 0)),
    out_shape=jax.ShapeDtypeStruct((B, 1, D), jnp.float32),
)

_proj = pl.pallas_call(
    _proj_body,
    grid=(1,),
    in_specs=[_full((B, D)), _full((D, D)), _full((1, D))],
    out_specs=_full((B, D)),
    out_shape=jax.ShapeDtypeStruct((B, D), jnp.float32),
)


def kernel(poi_table, dist_edges, dist_vec, data_x, data_poi,
           W0, b0, W1, b1, Wqkv, bqkv, Wo, bo):
  i32 = jnp.int32
  f32 = jnp.float32
  d0 = dist_edges[0].astype(i32)
  d1 = dist_edges[1].astype(i32)
  loop = jnp.arange(N, dtype=i32)
  epad = ETOT - (2 * E + N)
  pad_idx = jnp.full((epad,), NP - 1, i32)
  n1 = jnp.concatenate([d0, d1, loop, pad_idx])
  n2 = jnp.concatenate([d1, d0, loop, pad_idx])
  dv = jnp.concatenate([dist_vec.astype(f32), dist_vec.astype(f32),
                        jnp.zeros((N,), f32), jnp.full((epad,), 1e9, f32)])
  n1c = n1.reshape(ECHUNKS, CH)
  n2c = n2.reshape(ECHUNKS, CH)
  dv2 = dv.reshape(ECHUNKS, CH)
  nodes = jnp.concatenate(
      [d0, d1, jnp.full((HTOT - 2 * E,), NP - 1, i32)]).reshape(HTOT // 128, 128)

  degp = _hist(nodes).reshape(NC, NP)
  ptab = jnp.pad(poi_table.astype(f32), ((0, NP - N), (0, 0)))
  b0r = b0.reshape(1, D).astype(f32)
  b1r = b1.reshape(1, D).astype(f32)

  h1 = _mm_scale(ptab, degp, W0.astype(f32), b0r)
  p1 = _spmm(n1c, n2c, dv2, h1)
  h2 = _fuse(p1, p1, degp, W1.astype(f32), b1r)
  p2 = _spmm(n1c, n2c, dv2, h2)
  enc = _norm(p2, p2)

  gidx = jnp.concatenate([data_x.astype(i32), data_poi.astype(i32),
                          jnp.zeros((GTOT - B * L - B,), i32)])
  g = _gather(gidx, enc)
  poi_embed = g[B * L:B * L + B]
  seq3 = g[:B * L].reshape(B, L, D)
  ctxm = _attn(seq3, Wqkv.astype(f32),
               bqkv.reshape(1, 3 * D).astype(f32)).reshape(B, D)
  aggr = _proj(ctxm, Wo.astype(f32), bo.reshape(1, D).astype(f32))
  return (aggr, poi_embed)
